# baseline (speedup 1.0000x reference)
import functools

import jax
import jax.numpy as jnp
from jax import lax
from jax.experimental import pallas as pl
from jax.experimental.pallas import tpu as pltpu
from jax.experimental.pallas import tpu_sc as plsc

N = 10000
E = 320000
F = 128
CH = 128
NG = 32
NOUT = 10

NCORES = 2
NSUB = 16
NTILES = NCORES * NSUB
CHUNK = 64
NCHUNK = 160
EDGES_PER_TILE = NCHUNK * CHUNK
EPAD = NTILES * EDGES_PER_TILE
NPAD = 10240
ROWS_PER_TILE = NPAD // NSUB


def _sc_propagate(h, pk3, w3):
    mesh = plsc.VectorSubcoreMesh(core_axis_name="c", subcore_axis_name="s")

    @functools.partial(
        pl.kernel,
        out_type=jax.ShapeDtypeStruct((NCORES, NPAD, CH), jnp.float32),
        mesh=mesh,
        scratch_types=[
            pltpu.VMEM((CHUNK, CH), jnp.float32),
            pltpu.VMEM((CHUNK, CH), jnp.float32),
            pltpu.VMEM((CHUNK, CH), jnp.float32),
            pltpu.VMEM((CHUNK, CH), jnp.float32),
            pltpu.VMEM((2 * CHUNK,), jnp.int32),
            pltpu.VMEM((2 * CHUNK,), jnp.int32),
            pltpu.VMEM((CHUNK,), jnp.float32),
            pltpu.VMEM((CHUNK,), jnp.float32),
            pltpu.VMEM((CHUNK,), jnp.int32),
            pltpu.VMEM((CHUNK,), jnp.int32),
            pltpu.VMEM((CHUNK,), jnp.int32),
            pltpu.VMEM((CHUNK,), jnp.int32),
            pltpu.VMEM_SHARED((NPAD, CH), jnp.float32),
            pltpu.SemaphoreType.DMA,
            pltpu.SemaphoreType.DMA,
            pltpu.SemaphoreType.DMA,
            pltpu.SemaphoreType.DMA,
            pltpu.SemaphoreType.DMA,
            pltpu.SemaphoreType.DMA,
            pltpu.SemaphoreType.DMA,
        ],
    )
    def prop(h_hbm, pk_hbm, w_hbm, out_hbm,
             ga, gb, sa, sb, pk_x, pk_y, w_a, w_b, sidx_a, sidx_b,
             didx_a, didx_b, acc,
             sem_ga, sem_gb, sem_wa, sem_wb, sem_sa, sem_sb, sem_pky):
        cid = lax.axis_index("c")
        sid = lax.axis_index("s")
        tile = cid * NSUB + sid

        def stage_idx(idx, pkbuf, lane_off, shift):
            for gg in range(CHUNK // 16):
                sl = pl.ds(gg * 16, 16)
                v = pkbuf[pl.ds(lane_off + gg * 16, 16)]
                if shift:
                    idx[sl] = lax.shift_right_logical(v, 14)
                else:
                    idx[sl] = v & 16383

        def start_gather(gbuf, wbuf, sem_g, sem_w, sidx, g):
            pltpu.async_copy(h_hbm.at[sidx], gbuf, sem_g)
            pltpu.async_copy(w_hbm.at[tile, g], wbuf, sem_w)

        def wait_gather(gbuf, wbuf, sem_g, sem_w):
            pltpu.make_async_copy(h_hbm.at[pl.ds(0, CHUNK)], gbuf,
                                  sem_g).wait()
            pltpu.make_async_copy(w_hbm.at[0, 0], wbuf, sem_w).wait()

        def wait_scatter(sbuf, sem_s):
            pltpu.make_async_copy(h_hbm.at[pl.ds(0, CHUNK)], sbuf,
                                  sem_s).wait()

        def wait_pk(pkbuf, sem_pk):
            pltpu.make_async_copy(pk_hbm.at[0, 0], pkbuf, sem_pk).wait()

        def zero_buf(buf):
            @pl.loop(0, CHUNK)
            def _z(r):
                for j in range(CH // 16):
                    buf[r, pl.ds(j * 16, 16)] = jnp.zeros((16,), jnp.float32)

        pltpu.sync_copy(pk_hbm.at[tile, 0], pk_x)
        pltpu.async_copy(pk_hbm.at[tile, 1], pk_y, sem_pky)
        stage_idx(sidx_a, pk_x, 0, True)
        stage_idx(sidx_b, pk_x, CHUNK, True)
        start_gather(ga, w_a, sem_ga, sem_wa, sidx_a, 0)
        start_gather(gb, w_b, sem_gb, sem_wb, sidx_b, 1)
        zero_buf(sa)
        zero_buf(sb)
        for gg in range(CHUNK // 16):
            sl = pl.ds(gg * 16, 16)
            didx_a[sl] = jnp.zeros((16,), jnp.int32)
            didx_b[sl] = jnp.zeros((16,), jnp.int32)

        @pl.loop(0, ROWS_PER_TILE // CHUNK)
        def _zcopy(p_i):
            pltpu.sync_copy(
                sa, acc.at[pl.ds(sid * ROWS_PER_TILE + p_i * CHUNK, CHUNK)])

        plsc.subcore_barrier()

        def lane(gbuf, sbuf, wbuf, sidx, didx, sem_g, sem_w, sem_s,
                 cur, nxt, lane_off, g, pk_wait):
            wait_gather(gbuf, wbuf, sem_g, sem_w)
            stage_idx(didx, cur, lane_off, False)

            @pl.loop(0, CHUNK // 16)
            def _scale(gg):
                wvec = wbuf[pl.ds(gg * 16, 16)]
                for t in range(16):
                    e = gg * 16 + t
                    wv = jnp.full((16,), wvec[t], dtype=jnp.float32)
                    for j in range(CH // 16):
                        sl = pl.ds(j * 16, 16)
                        sbuf[e, sl] = gbuf[e, sl] * wv

            pltpu.sync_copy(sbuf, acc.at[didx], add=True)
            if pk_wait:
                wait_pk(nxt, sem_pky)
            stage_idx(sidx, nxt, lane_off, True)
            start_gather(gbuf, wbuf, sem_g, sem_w, sidx,
                         jnp.minimum(g + 2, NCHUNK - 1))

        def pair(cur, nxt, g0):
            lane(ga, sa, w_a, sidx_a, didx_a, sem_ga, sem_wa, sem_sa,
                 cur, nxt, 0, g0, True)
            lane(gb, sb, w_b, sidx_b, didx_b, sem_gb, sem_wb, sem_sb,
                 cur, nxt, CHUNK, g0 + 1, False)
            pltpu.async_copy(
                pk_hbm.at[tile, jnp.minimum(g0 // 2 + 2, NCHUNK // 2 - 1)],
                cur, sem_pky)

        @pl.loop(0, NCHUNK // 4)
        def _quad(q):
            pair(pk_x, pk_y, 4 * q)
            pair(pk_y, pk_x, 4 * q + 2)

        wait_pk(pk_y, sem_pky)
        wait_gather(ga, w_a, sem_ga, sem_wa)
        wait_gather(gb, w_b, sem_gb, sem_wb)
        plsc.subcore_barrier()
        pltpu.sync_copy(
            acc.at[pl.ds(sid * ROWS_PER_TILE, ROWS_PER_TILE)],
            out_hbm.at[cid, pl.ds(sid * ROWS_PER_TILE, ROWS_PER_TILE)])

    return prop(h, pk3, w3)


RB = 1000
NRB = N // RB


def _elu(v):
    return jnp.where(v > 0, v, jnp.exp(v) - 1.0)


def _mm2_body(x_ref, k1_ref, k2_ref, h_ref, s_ref):
    xb = x_ref[...]
    h_ref[...] = jnp.dot(xb, k1_ref[...], preferred_element_type=jnp.float32)
    s_ref[...] = jnp.dot(xb, k2_ref[...], preferred_element_type=jnp.float32)


def _mm2(x, k1, k2):
    return pl.pallas_call(
        _mm2_body,
        grid=(NRB,),
        in_specs=[
            pl.BlockSpec((RB, F), lambda i: (i, 0)),
            pl.BlockSpec((F, CH), lambda i: (0, 0)),
            pl.BlockSpec((F, CH), lambda i: (0, 0)),
        ],
        out_specs=[
            pl.BlockSpec((RB, CH), lambda i: (i, 0)),
            pl.BlockSpec((RB, CH), lambda i: (i, 0)),
        ],
        out_shape=[
            jax.ShapeDtypeStruct((N, CH), jnp.float32),
            jax.ShapeDtypeStruct((N, CH), jnp.float32),
        ],
    )(x, k1, k2)


def _combine_mm2_body(p0_ref, p1_ref, s_ref, b_ref, k1_ref, k2_ref,
                      h_ref, s2_ref):
    out = _elu(_elu(p0_ref[...] + p1_ref[...] + s_ref[...] + b_ref[...]))
    h_ref[...] = jnp.dot(out, k1_ref[...], preferred_element_type=jnp.float32)
    s2_ref[...] = jnp.dot(out, k2_ref[...], preferred_element_type=jnp.float32)


def _combine_mm2(p0, p1, s, b, k1, k2):
    return pl.pallas_call(
        _combine_mm2_body,
        grid=(NRB,),
        in_specs=[
            pl.BlockSpec((RB, CH), lambda i: (i, 0)),
            pl.BlockSpec((RB, CH), lambda i: (i, 0)),
            pl.BlockSpec((RB, CH), lambda i: (i, 0)),
            pl.BlockSpec((1, CH), lambda i: (0, 0)),
            pl.BlockSpec((CH, CH), lambda i: (0, 0)),
            pl.BlockSpec((CH, CH), lambda i: (0, 0)),
        ],
        out_specs=[
            pl.BlockSpec((RB, CH), lambda i: (i, 0)),
            pl.BlockSpec((RB, CH), lambda i: (i, 0)),
        ],
        out_shape=[
            jax.ShapeDtypeStruct((N, CH), jnp.float32),
            jax.ShapeDtypeStruct((N, CH), jnp.float32),
        ],
    )(p0, p1, s, b, k1, k2)


def _head_body(p0_ref, p1_ref, s_ref, b_ref, gid_ref, d1w_ref, d1b_ref,
               d2w_ref, d2b_ref, out_ref, pooled_ref, cnt_ref):
    i = pl.program_id(0)

    @pl.when(i == 0)
    def _init():
        pooled_ref[...] = jnp.zeros((NG, CH), jnp.float32)
        cnt_ref[...] = jnp.zeros((NG, CH), jnp.float32)

    out2 = _elu(_elu(p0_ref[...] + p1_ref[...] + s_ref[...] + b_ref[...]))
    gids = gid_ref[0, 0, :]
    onehot = (gids[None, :] == lax.broadcasted_iota(jnp.int32, (NG, RB), 0)
              ).astype(jnp.float32)
    pooled_ref[...] += jnp.dot(onehot, out2,
                               preferred_element_type=jnp.float32)
    cnt_ref[...] += jnp.dot(onehot, jnp.ones((RB, CH), jnp.float32),
                            preferred_element_type=jnp.float32)

    @pl.when(i == NRB - 1)
    def _finish():
        pooled = pooled_ref[...] / jnp.maximum(cnt_ref[...], 1.0)
        d1 = jnp.maximum(
            jnp.dot(pooled, d1w_ref[...], preferred_element_type=jnp.float32)
            + d1b_ref[...], 0.0)
        logits = jnp.dot(d1, d2w_ref[...],
                         preferred_element_type=jnp.float32) + d2b_ref[...]
        z = logits - jnp.max(logits, axis=-1, keepdims=True)
        ez = jnp.exp(z)
        out_ref[...] = ez / jnp.sum(ez, axis=-1, keepdims=True)


def _head(p0, p1, s, b, gids3, d1w, d1b, d2w, d2b):
    return pl.pallas_call(
        _head_body,
        grid=(NRB,),
        in_specs=[
            pl.BlockSpec((RB, CH), lambda i: (i, 0)),
            pl.BlockSpec((RB, CH), lambda i: (i, 0)),
            pl.BlockSpec((RB, CH), lambda i: (i, 0)),
            pl.BlockSpec((1, CH), lambda i: (0, 0)),
            pl.BlockSpec((1, 1, RB), lambda i: (i, 0, 0)),
            pl.BlockSpec((CH, CH), lambda i: (0, 0)),
            pl.BlockSpec((1, CH), lambda i: (0, 0)),
            pl.BlockSpec((CH, NOUT), lambda i: (0, 0)),
            pl.BlockSpec((1, NOUT), lambda i: (0, 0)),
        ],
        out_specs=pl.BlockSpec((NG, NOUT), lambda i: (0, 0)),
        out_shape=jax.ShapeDtypeStruct((NG, NOUT), jnp.float32),
        scratch_shapes=[
            pltpu.VMEM((NG, CH), jnp.float32),
            pltpu.VMEM((NG, CH), jnp.float32),
        ],
    )(p0, p1, s, b, gids3, d1w, d1b, d2w, d2b)


def kernel(x, edge_weight, conv1_k1, conv1_k2, conv1_b, conv2_k1, conv2_k2,
           conv2_b, dense1_w, dense1_b, dense2_w, dense2_b, edge_index,
           graph_ids):
    pk = edge_index[0] * 16384 + edge_index[1]
    pk3 = jnp.pad(pk, (0, EPAD - E)).reshape(NTILES, NCHUNK // 2, 2 * CHUNK)
    w3 = jnp.pad(edge_weight, (0, EPAD - E)).reshape(NTILES, NCHUNK, CHUNK)

    h1, s1 = _mm2(x, conv1_k1, conv1_k2)
    p1 = _sc_propagate(h1, pk3, w3)[:, :N]
    h2, s2 = _combine_mm2(p1[0], p1[1], s1, conv1_b.reshape(1, CH),
                          conv2_k1, conv2_k2)
    p2 = _sc_propagate(h2, pk3, w3)[:, :N]
    gids3 = graph_ids.reshape(NRB, 1, RB)
    return _head(p2[0], p2[1], s2, conv2_b.reshape(1, CH), gids3,
                 dense1_w, dense1_b.reshape(1, CH),
                 dense2_w, dense2_b.reshape(1, NOUT))


# speedup vs baseline: 1.0567x; 1.0567x over previous
import functools

import jax
import jax.numpy as jnp
from jax import lax
from jax.experimental import pallas as pl
from jax.experimental.pallas import tpu as pltpu
from jax.experimental.pallas import tpu_sc as plsc

N = 10000
E = 320000
F = 128
CH = 128
NG = 32
NOUT = 10

NCORES = 2
NSUB = 16
NTILES = NCORES * NSUB
CHUNK = 64
NCHUNK = 160
EDGES_PER_TILE = NCHUNK * CHUNK
EPAD = NTILES * EDGES_PER_TILE
NPAD = 10240
ROWS_PER_TILE = NPAD // NSUB


BLK = 16
NBLK = NCHUNK // BLK


def _sc_propagate(h, pk4, w4):
    mesh = plsc.VectorSubcoreMesh(core_axis_name="c", subcore_axis_name="s")

    @functools.partial(
        pl.kernel,
        out_type=jax.ShapeDtypeStruct((NCORES, NPAD, CH), jnp.float32),
        mesh=mesh,
        scratch_types=[
            pltpu.VMEM((CHUNK, CH), jnp.float32),
            pltpu.VMEM((CHUNK, CH), jnp.float32),
            pltpu.VMEM((CHUNK, CH), jnp.float32),
            pltpu.VMEM((CHUNK, CH), jnp.float32),
            pltpu.VMEM((2, BLK, CHUNK), jnp.int32),
            pltpu.VMEM((2, BLK, CHUNK), jnp.float32),
            pltpu.VMEM((CHUNK,), jnp.int32),
            pltpu.VMEM((CHUNK,), jnp.int32),
            pltpu.VMEM((CHUNK,), jnp.int32),
            pltpu.VMEM((CHUNK,), jnp.int32),
            pltpu.VMEM_SHARED((NPAD, CH), jnp.float32),
            pltpu.SemaphoreType.DMA,
            pltpu.SemaphoreType.DMA,
            pltpu.SemaphoreType.DMA,
            pltpu.SemaphoreType.DMA,
            pltpu.SemaphoreType.DMA,
        ],
    )
    def prop(h_hbm, pk_hbm, w_hbm, out_hbm,
             ga, gb, sa, sb, mblk, wblk, sidx_a, sidx_b, didx_a, didx_b,
             acc, sem_ga, sem_gb, sem_sa, sem_sb, sem_m):
        cid = lax.axis_index("c")
        sid = lax.axis_index("s")
        tile = cid * NSUB + sid

        def wait_rows(buf, sem):
            pltpu.make_async_copy(h_hbm.at[pl.ds(0, CHUNK)], buf,
                                  sem).wait()

        def wait_meta():
            pltpu.make_async_copy(pk_hbm.at[0, 0], mblk.at[0], sem_m).wait()
            pltpu.make_async_copy(w_hbm.at[0, 0], wblk.at[0], sem_m).wait()

        def load_block(blk, slot):
            pltpu.async_copy(pk_hbm.at[tile, blk], mblk.at[slot], sem_m)
            pltpu.async_copy(w_hbm.at[tile, blk], wblk.at[slot], sem_m)

        def stage_sidx(sidx, g):
            q = (g // BLK) % 2
            m = g % BLK
            for gg in range(CHUNK // 16):
                sl = pl.ds(gg * 16, 16)
                sidx[sl] = lax.shift_right_logical(mblk[q, m, sl], 14)

        def zero_buf(buf):
            @pl.loop(0, CHUNK)
            def _z(r):
                for j in range(CH // 16):
                    buf[r, pl.ds(j * 16, 16)] = jnp.zeros((16,), jnp.float32)

        load_block(0, 0)
        load_block(1, 1)
        wait_meta()
        wait_meta()
        stage_sidx(sidx_a, 0)
        stage_sidx(sidx_b, 1)
        pltpu.async_copy(h_hbm.at[sidx_a], ga, sem_ga)
        pltpu.async_copy(h_hbm.at[sidx_b], gb, sem_gb)
        zero_buf(sa)
        zero_buf(sb)
        for gg in range(CHUNK // 16):
            sl = pl.ds(gg * 16, 16)
            didx_a[sl] = jnp.zeros((16,), jnp.int32)
            didx_b[sl] = jnp.zeros((16,), jnp.int32)

        @pl.loop(0, ROWS_PER_TILE // CHUNK)
        def _zcopy(p_i):
            pltpu.sync_copy(
                sa, acc.at[pl.ds(sid * ROWS_PER_TILE + p_i * CHUNK, CHUNK)])

        pltpu.async_copy(sa, acc.at[didx_a], sem_sa, add=True)
        pltpu.async_copy(sb, acc.at[didx_b], sem_sb, add=True)
        plsc.subcore_barrier()

        def lane(gbuf, sbuf, sidx, didx, sem_g, sem_s, g):
            wait_rows(gbuf, sem_g)
            wait_rows(sbuf, sem_s)
            q = (g // BLK) % 2
            m = g % BLK
            for gg in range(CHUNK // 16):
                sl = pl.ds(gg * 16, 16)
                didx[sl] = mblk[q, m, sl] & 16383

            @pl.loop(0, CHUNK // 16)
            def _scale(gg):
                wvec = wblk[q, m, pl.ds(gg * 16, 16)]
                for t in range(16):
                    e = gg * 16 + t
                    wv = jnp.full((16,), wvec[t], dtype=jnp.float32)
                    for j in range(CH // 16):
                        sl = pl.ds(j * 16, 16)
                        sbuf[e, sl] = gbuf[e, sl] * wv

            pltpu.async_copy(sbuf, acc.at[didx], sem_s, add=True)
            stage_sidx(sidx, jnp.minimum(g + 2, NCHUNK - 1))
            pltpu.async_copy(h_hbm.at[sidx], gbuf, sem_g)

        @pl.loop(0, NCHUNK // 2)
        def _pair(p):
            g0 = 2 * p
            last_of_block = g0 % BLK == BLK - 2

            @pl.when(jnp.logical_and(last_of_block, g0 >= BLK))
            def _w():
                wait_meta()

            lane(ga, sa, sidx_a, didx_a, sem_ga, sem_sa, g0)
            lane(gb, sb, sidx_b, didx_b, sem_gb, sem_sb, g0 + 1)

            @pl.when(last_of_block)
            def _l():
                load_block(jnp.minimum(g0 // BLK + 2, NBLK - 1),
                           (g0 // BLK) % 2)

        wait_meta()
        wait_rows(sa, sem_sa)
        wait_rows(sb, sem_sb)
        wait_rows(ga, sem_ga)
        wait_rows(gb, sem_gb)
        plsc.subcore_barrier()
        pltpu.sync_copy(
            acc.at[pl.ds(sid * ROWS_PER_TILE, ROWS_PER_TILE)],
            out_hbm.at[cid, pl.ds(sid * ROWS_PER_TILE, ROWS_PER_TILE)])

    return prop(h, pk4, w4)


RB = 1000
NRB = N // RB


def _elu(v):
    return jnp.where(v > 0, v, jnp.exp(v) - 1.0)


def _mm2_body(x_ref, k1_ref, k2_ref, h_ref, s_ref):
    xb = x_ref[...]
    h_ref[...] = jnp.dot(xb, k1_ref[...], preferred_element_type=jnp.float32)
    s_ref[...] = jnp.dot(xb, k2_ref[...], preferred_element_type=jnp.float32)


def _mm2(x, k1, k2):
    return pl.pallas_call(
        _mm2_body,
        grid=(NRB,),
        in_specs=[
            pl.BlockSpec((RB, F), lambda i: (i, 0)),
            pl.BlockSpec((F, CH), lambda i: (0, 0)),
            pl.BlockSpec((F, CH), lambda i: (0, 0)),
        ],
        out_specs=[
            pl.BlockSpec((RB, CH), lambda i: (i, 0)),
            pl.BlockSpec((RB, CH), lambda i: (i, 0)),
        ],
        out_shape=[
            jax.ShapeDtypeStruct((N, CH), jnp.float32),
            jax.ShapeDtypeStruct((N, CH), jnp.float32),
        ],
    )(x, k1, k2)


def _combine_mm2_body(p0_ref, p1_ref, s_ref, b_ref, k1_ref, k2_ref,
                      h_ref, s2_ref):
    out = _elu(_elu(p0_ref[...] + p1_ref[...] + s_ref[...] + b_ref[...]))
    h_ref[...] = jnp.dot(out, k1_ref[...], preferred_element_type=jnp.float32)
    s2_ref[...] = jnp.dot(out, k2_ref[...], preferred_element_type=jnp.float32)


def _combine_mm2(p0, p1, s, b, k1, k2):
    return pl.pallas_call(
        _combine_mm2_body,
        grid=(NRB,),
        in_specs=[
            pl.BlockSpec((RB, CH), lambda i: (i, 0)),
            pl.BlockSpec((RB, CH), lambda i: (i, 0)),
            pl.BlockSpec((RB, CH), lambda i: (i, 0)),
            pl.BlockSpec((1, CH), lambda i: (0, 0)),
            pl.BlockSpec((CH, CH), lambda i: (0, 0)),
            pl.BlockSpec((CH, CH), lambda i: (0, 0)),
        ],
        out_specs=[
            pl.BlockSpec((RB, CH), lambda i: (i, 0)),
            pl.BlockSpec((RB, CH), lambda i: (i, 0)),
        ],
        out_shape=[
            jax.ShapeDtypeStruct((N, CH), jnp.float32),
            jax.ShapeDtypeStruct((N, CH), jnp.float32),
        ],
    )(p0, p1, s, b, k1, k2)


def _head_body(p0_ref, p1_ref, s_ref, b_ref, gid_ref, d1w_ref, d1b_ref,
               d2w_ref, d2b_ref, out_ref, pooled_ref, cnt_ref):
    i = pl.program_id(0)

    @pl.when(i == 0)
    def _init():
        pooled_ref[...] = jnp.zeros((NG, CH), jnp.float32)
        cnt_ref[...] = jnp.zeros((NG, CH), jnp.float32)

    out2 = _elu(_elu(p0_ref[...] + p1_ref[...] + s_ref[...] + b_ref[...]))
    gids = gid_ref[0, 0, :]
    onehot = (gids[None, :] == lax.broadcasted_iota(jnp.int32, (NG, RB), 0)
              ).astype(jnp.float32)
    pooled_ref[...] += jnp.dot(onehot, out2,
                               preferred_element_type=jnp.float32)
    cnt_ref[...] += jnp.dot(onehot, jnp.ones((RB, CH), jnp.float32),
                            preferred_element_type=jnp.float32)

    @pl.when(i == NRB - 1)
    def _finish():
        pooled = pooled_ref[...] / jnp.maximum(cnt_ref[...], 1.0)
        d1 = jnp.maximum(
            jnp.dot(pooled, d1w_ref[...], preferred_element_type=jnp.float32)
            + d1b_ref[...], 0.0)
        logits = jnp.dot(d1, d2w_ref[...],
                         preferred_element_type=jnp.float32) + d2b_ref[...]
        z = logits - jnp.max(logits, axis=-1, keepdims=True)
        ez = jnp.exp(z)
        out_ref[...] = ez / jnp.sum(ez, axis=-1, keepdims=True)


def _head(p0, p1, s, b, gids3, d1w, d1b, d2w, d2b):
    return pl.pallas_call(
        _head_body,
        grid=(NRB,),
        in_specs=[
            pl.BlockSpec((RB, CH), lambda i: (i, 0)),
            pl.BlockSpec((RB, CH), lambda i: (i, 0)),
            pl.BlockSpec((RB, CH), lambda i: (i, 0)),
            pl.BlockSpec((1, CH), lambda i: (0, 0)),
            pl.BlockSpec((1, 1, RB), lambda i: (i, 0, 0)),
            pl.BlockSpec((CH, CH), lambda i: (0, 0)),
            pl.BlockSpec((1, CH), lambda i: (0, 0)),
            pl.BlockSpec((CH, NOUT), lambda i: (0, 0)),
            pl.BlockSpec((1, NOUT), lambda i: (0, 0)),
        ],
        out_specs=pl.BlockSpec((NG, NOUT), lambda i: (0, 0)),
        out_shape=jax.ShapeDtypeStruct((NG, NOUT), jnp.float32),
        scratch_shapes=[
            pltpu.VMEM((NG, CH), jnp.float32),
            pltpu.VMEM((NG, CH), jnp.float32),
        ],
    )(p0, p1, s, b, gids3, d1w, d1b, d2w, d2b)


def kernel(x, edge_weight, conv1_k1, conv1_k2, conv1_b, conv2_k1, conv2_k2,
           conv2_b, dense1_w, dense1_b, dense2_w, dense2_b, edge_index,
           graph_ids):
    pk = edge_index[0] * 16384 + edge_index[1]
    pk4 = jnp.pad(pk, (0, EPAD - E)).reshape(NTILES, NBLK, BLK, CHUNK)
    w4 = jnp.pad(edge_weight, (0, EPAD - E)).reshape(NTILES, NBLK, BLK, CHUNK)

    h1, s1 = _mm2(x, conv1_k1, conv1_k2)
    p1 = _sc_propagate(h1, pk4, w4)[:, :N]
    h2, s2 = _combine_mm2(p1[0], p1[1], s1, conv1_b.reshape(1, CH),
                          conv2_k1, conv2_k2)
    p2 = _sc_propagate(h2, pk4, w4)[:, :N]
    gids3 = graph_ids.reshape(NRB, 1, RB)
    return _head(p2[0], p2[1], s2, conv2_b.reshape(1, CH), gids3,
                 dense1_w, dense1_b.reshape(1, CH),
                 dense2_w, dense2_b.reshape(1, NOUT))
